# baseline (speedup 1.0000x reference)
import functools

import jax
import jax.numpy as jnp
from jax import lax
from jax.experimental import pallas as pl
from jax.experimental.pallas import tpu as pltpu
from jax.experimental.pallas import tpu_sc as plsc

NUM_TOKENS = 4096
MODEL_DIM = 1024
NUM_EXPERTS = 64
CAPACITY = 2 * NUM_TOKENS // NUM_EXPERTS
TBLK = 256
NB = NUM_TOKENS // TBLK
CHUNK = 128

ROW_WORDS = NUM_EXPERTS * CAPACITY
OUT_WORDS = NUM_TOKENS * ROW_WORDS
SC_WORKERS = 32
TOK_PER_W = NUM_TOKENS // SC_WORKERS
ENT_PER_W = TOK_PER_W * 2
CHUNK_TOK = 8
CHUNK_WORDS = CHUNK_TOK * ROW_WORDS
N_CHUNKS = TOK_PER_W // CHUNK_TOK


def _cumsum_tokens(m):
    nb = NUM_TOKENS // CHUNK
    mb = m.reshape(nb, CHUNK, NUM_EXPERTS)
    ii = lax.broadcasted_iota(jnp.int32, (CHUNK, CHUNK), 0)
    jj = lax.broadcasted_iota(jnp.int32, (CHUNK, CHUNK), 1)
    tri = (jj <= ii).astype(jnp.float32)
    trib = jnp.broadcast_to(tri, (nb, CHUNK, CHUNK))
    within = lax.dot_general(
        trib, mb, (((2,), (1,)), ((0,), (0,))),
        preferred_element_type=jnp.float32)
    sums = within[:, CHUNK - 1, :]
    bi = lax.broadcasted_iota(jnp.int32, (nb, nb), 0)
    bj = lax.broadcasted_iota(jnp.int32, (nb, nb), 1)
    tri_x = (bj < bi).astype(jnp.float32)
    carry = lax.dot_general(
        tri_x, sums, (((1,), (0,)), ((), ())),
        preferred_element_type=jnp.float32)
    return (within + carry[:, None, :]).reshape(NUM_TOKENS, NUM_EXPERTS)


def _route_kernel(x_ref, w_ref, noise_ref,
                  laux_ref, le8_ref, offs_ref, vals_ref):
    x = x_ref[...]
    w = w_ref[...]
    logits = lax.dot_general(
        x, w, (((1,), (1,)), ((), ())),
        preferred_element_type=jnp.float32)
    m = jnp.max(logits, axis=1, keepdims=True)
    ex = jnp.exp(logits - m)
    gates = ex / jnp.sum(ex, axis=1, keepdims=True)
    iota_e = lax.broadcasted_iota(jnp.int32, (NUM_TOKENS, NUM_EXPERTS), 1)
    e1 = jnp.argmax(logits, axis=1)
    oh1 = iota_e == e1[:, None]
    g1 = jnp.max(gates, axis=1)
    lx = jnp.where(oh1, -jnp.inf, logits + noise_ref[...])
    e2 = jnp.argmax(lx, axis=1)
    oh2 = iota_e == e2[:, None]
    g2 = jnp.sum(jnp.where(oh2, gates, 0.0), axis=1)

    cs1 = _cumsum_tokens(oh1.astype(jnp.float32))
    cs2 = _cumsum_tokens(oh2.astype(jnp.float32))
    counts1 = cs1[NUM_TOKENS - 1:NUM_TOKENS, :]
    loc1 = jnp.sum(jnp.where(oh1, cs1 - 1.0, 0.0), axis=1)
    loc2 = jnp.sum(jnp.where(oh2, cs2 - 1.0 + counts1, 0.0), axis=1)

    g1s = jnp.where(loc1 < CAPACITY, g1, 0.0)
    g2s = jnp.where(loc2 < CAPACITY, g2, 0.0)
    denom = jnp.maximum(g1s + g2s, jnp.finfo(jnp.float32).eps)
    g1n = g1s / denom
    g2n = g2s / denom

    loc1i = loc1.astype(jnp.int32)
    loc2i = loc2.astype(jnp.int32)
    m1 = oh1 & ((loc1i < CAPACITY) & (g1n != 0.0))[:, None]
    m2 = oh2 & ((loc2i < CAPACITY) & (g2n != 0.0))[:, None]
    le = jnp.where(m1, loc1i[:, None],
                   jnp.where(m2, loc2i[:, None], -1))
    le8_ref[...] = le.astype(jnp.int8)

    off1 = e1.astype(jnp.int32) * CAPACITY + jnp.minimum(loc1i, CAPACITY - 1)
    off2 = e2.astype(jnp.int32) * CAPACITY + jnp.minimum(loc2i, CAPACITY - 1)
    offs_ref[...] = jnp.concatenate(
        [off1[:, None], off2[:, None]], axis=1)
    vals_ref[...] = jnp.concatenate(
        [g1n[:, None], g2n[:, None]], axis=1)

    me_sum = jnp.sum(gates, axis=0, keepdims=True)
    laux_ref[...] = jnp.sum(me_sum * counts1, axis=1, keepdims=True) / (
        float(NUM_EXPERTS) * NUM_TOKENS * NUM_TOKENS)


def _disp_kernel(le8_ref, disp_ref):
    le8 = le8_ref[...]
    iota_c = lax.broadcasted_iota(jnp.int8, (TBLK, CAPACITY), 1)
    for e in range(NUM_EXPERTS):
        disp_ref[:, pl.ds(e * CAPACITY, CAPACITY)] = (
            iota_c == le8[:, e:e + 1])


@functools.lru_cache(maxsize=1)
def _sc_combine_fn():
    return functools.partial(
        pl.kernel,
        out_type=jax.ShapeDtypeStruct((OUT_WORDS,), jnp.float32),
        mesh=plsc.VectorSubcoreMesh(core_axis_name="c",
                                    subcore_axis_name="s"),
        compiler_params=pltpu.CompilerParams(needs_layout_passes=False),
        scratch_types=[
            pltpu.VMEM((ENT_PER_W,), jnp.int32),
            pltpu.VMEM((ENT_PER_W,), jnp.float32),
            pltpu.VMEM((CHUNK_WORDS,), jnp.float32),
        ],
    )(_sc_combine_body)


def _sc_combine_body(offs_hbm, vals_hbm, out_hbm, offs_v, vals_v, ring):
    w = lax.axis_index("s") * 2 + lax.axis_index("c")
    pltpu.sync_copy(offs_hbm.at[pl.ds(w * ENT_PER_W, ENT_PER_W)], offs_v)
    pltpu.sync_copy(vals_hbm.at[pl.ds(w * ENT_PER_W, ENT_PER_W)], vals_v)

    zero16 = jnp.zeros((16,), jnp.float32)

    def _zero_body(i, c):
        ring[pl.ds(i * 16, 16)] = zero16
        return c

    lax.fori_loop(0, CHUNK_WORDS // 16, _zero_body, 0)

    pattern = lax.shift_left(
        lax.shift_right_logical(lax.iota(jnp.int32, 16), 1), 13)
    out_base = w * (TOK_PER_W * ROW_WORDS)

    def _body(j, c):
        idx = offs_v[pl.ds(j * 16, 16)] + pattern
        val = vals_v[pl.ds(j * 16, 16)]
        plsc.store_scatter(ring, [idx], val)
        pltpu.sync_copy(
            ring,
            out_hbm.at[pl.ds(out_base + j * CHUNK_WORDS, CHUNK_WORDS)])
        plsc.store_scatter(ring, [idx], zero16)
        return c

    lax.fori_loop(0, N_CHUNKS, _body, 0)


def kernel(input, W):
    noise = jax.random.gumbel(
        jax.random.key(42), (NUM_TOKENS, NUM_EXPERTS), dtype=jnp.float32)
    laux, le8, offs, vals = pl.pallas_call(
        _route_kernel,
        out_shape=[
            jax.ShapeDtypeStruct((1, 1), jnp.float32),
            jax.ShapeDtypeStruct((NUM_TOKENS, NUM_EXPERTS), jnp.int8),
            jax.ShapeDtypeStruct((NUM_TOKENS, 2), jnp.int32),
            jax.ShapeDtypeStruct((NUM_TOKENS, 2), jnp.float32),
        ],
    )(input, W, noise)
    cw_flat = _sc_combine_fn()(offs.reshape(-1), vals.reshape(-1))
    disp = pl.pallas_call(
        _disp_kernel,
        grid=(NB,),
        in_specs=[
            pl.BlockSpec((TBLK, NUM_EXPERTS), lambda i: (i, 0)),
        ],
        out_specs=[
            pl.BlockSpec((TBLK, ROW_WORDS), lambda i: (i, 0)),
        ],
        out_shape=[
            jax.ShapeDtypeStruct((NUM_TOKENS, ROW_WORDS), jnp.bool_),
        ],
    )(le8)[0]
    cw = cw_flat.reshape(NUM_TOKENS, NUM_EXPERTS, CAPACITY)
    return laux.reshape(()), cw, disp.reshape(
        NUM_TOKENS, NUM_EXPERTS, CAPACITY)


# speedup vs baseline: 1.4082x; 1.4082x over previous
import functools

import jax
import jax.numpy as jnp
from jax import lax
from jax.experimental import pallas as pl
from jax.experimental.pallas import tpu as pltpu
from jax.experimental.pallas import tpu_sc as plsc

NUM_TOKENS = 4096
MODEL_DIM = 1024
NUM_EXPERTS = 64
CAPACITY = 2 * NUM_TOKENS // NUM_EXPERTS
TBLK = 256
NB = NUM_TOKENS // TBLK
CHUNK = 128

ROW_WORDS = NUM_EXPERTS * CAPACITY
OUT_WORDS = NUM_TOKENS * ROW_WORDS
SC_WORKERS = 32
TOK_PER_W = NUM_TOKENS // SC_WORKERS
ENT_PER_W = TOK_PER_W * 2
CHUNK_TOK = 8
CHUNK_WORDS = CHUNK_TOK * ROW_WORDS
N_CHUNKS = TOK_PER_W // CHUNK_TOK


def _cumsum_tokens(m):
    nb = NUM_TOKENS // CHUNK
    mb = m.reshape(nb, CHUNK, NUM_EXPERTS)
    ii = lax.broadcasted_iota(jnp.int32, (CHUNK, CHUNK), 0)
    jj = lax.broadcasted_iota(jnp.int32, (CHUNK, CHUNK), 1)
    tri = (jj <= ii).astype(jnp.float32)
    trib = jnp.broadcast_to(tri, (nb, CHUNK, CHUNK))
    within = lax.dot_general(
        trib, mb, (((2,), (1,)), ((0,), (0,))),
        preferred_element_type=jnp.float32)
    sums = within[:, CHUNK - 1, :]
    bi = lax.broadcasted_iota(jnp.int32, (nb, nb), 0)
    bj = lax.broadcasted_iota(jnp.int32, (nb, nb), 1)
    tri_x = (bj < bi).astype(jnp.float32)
    carry = lax.dot_general(
        tri_x, sums, (((1,), (0,)), ((), ())),
        preferred_element_type=jnp.float32)
    return (within + carry[:, None, :]).reshape(NUM_TOKENS, NUM_EXPERTS)


def _route_kernel(x_ref, w_ref, noise_ref,
                  laux_ref, le8_ref, offs_ref, vals_ref):
    x = x_ref[...]
    w = w_ref[...]
    logits = lax.dot_general(
        x, w, (((1,), (1,)), ((), ())),
        preferred_element_type=jnp.float32)
    m = jnp.max(logits, axis=1, keepdims=True)
    ex = jnp.exp(logits - m)
    gates = ex / jnp.sum(ex, axis=1, keepdims=True)
    iota_e = lax.broadcasted_iota(jnp.int32, (NUM_TOKENS, NUM_EXPERTS), 1)
    e1 = jnp.argmax(logits, axis=1)
    oh1 = iota_e == e1[:, None]
    g1 = jnp.max(gates, axis=1)
    lx = jnp.where(oh1, -jnp.inf, logits + noise_ref[...])
    e2 = jnp.argmax(lx, axis=1)
    oh2 = iota_e == e2[:, None]
    g2 = jnp.sum(jnp.where(oh2, gates, 0.0), axis=1)

    cs1 = _cumsum_tokens(oh1.astype(jnp.float32))
    cs2 = _cumsum_tokens(oh2.astype(jnp.float32))
    counts1 = cs1[NUM_TOKENS - 1:NUM_TOKENS, :]
    loc1 = jnp.sum(jnp.where(oh1, cs1 - 1.0, 0.0), axis=1)
    loc2 = jnp.sum(jnp.where(oh2, cs2 - 1.0 + counts1, 0.0), axis=1)

    g1s = jnp.where(loc1 < CAPACITY, g1, 0.0)
    g2s = jnp.where(loc2 < CAPACITY, g2, 0.0)
    denom = jnp.maximum(g1s + g2s, jnp.finfo(jnp.float32).eps)
    g1n = g1s / denom
    g2n = g2s / denom

    loc1i = loc1.astype(jnp.int32)
    loc2i = loc2.astype(jnp.int32)
    m1 = oh1 & ((loc1i < CAPACITY) & (g1n != 0.0))[:, None]
    m2 = oh2 & ((loc2i < CAPACITY) & (g2n != 0.0))[:, None]
    le = jnp.where(m1, loc1i[:, None],
                   jnp.where(m2, loc2i[:, None], -1))
    le8_ref[...] = le.astype(jnp.int8)

    off1 = e1.astype(jnp.int32) * CAPACITY + jnp.minimum(loc1i, CAPACITY - 1)
    off2 = e2.astype(jnp.int32) * CAPACITY + jnp.minimum(loc2i, CAPACITY - 1)
    offs_ref[...] = jnp.concatenate(
        [off1[:, None], off2[:, None]], axis=1)
    vals_ref[...] = jnp.concatenate(
        [g1n[:, None], g2n[:, None]], axis=1)

    me_sum = jnp.sum(gates, axis=0, keepdims=True)
    laux_ref[...] = jnp.sum(me_sum * counts1, axis=1, keepdims=True) / (
        float(NUM_EXPERTS) * NUM_TOKENS * NUM_TOKENS)


def _disp_kernel(le8_ref, disp_ref):
    le8 = le8_ref[...]
    iota_c = lax.broadcasted_iota(
        jnp.int8, (TBLK, NUM_EXPERTS, CAPACITY), 2)
    disp_ref[...] = iota_c == le8[:, :, None]


@functools.lru_cache(maxsize=1)
def _sc_combine_fn():
    return functools.partial(
        pl.kernel,
        out_type=jax.ShapeDtypeStruct((OUT_WORDS,), jnp.float32),
        mesh=plsc.VectorSubcoreMesh(core_axis_name="c",
                                    subcore_axis_name="s"),
        compiler_params=pltpu.CompilerParams(needs_layout_passes=False),
        scratch_types=[
            pltpu.VMEM((ENT_PER_W,), jnp.int32),
            pltpu.VMEM((ENT_PER_W,), jnp.float32),
            pltpu.VMEM((CHUNK_WORDS,), jnp.float32),
        ],
    )(_sc_combine_body)


def _sc_combine_body(offs_hbm, vals_hbm, out_hbm, offs_v, vals_v, ring):
    w = lax.axis_index("s") * 2 + lax.axis_index("c")
    pltpu.sync_copy(offs_hbm.at[pl.ds(w * ENT_PER_W, ENT_PER_W)], offs_v)
    pltpu.sync_copy(vals_hbm.at[pl.ds(w * ENT_PER_W, ENT_PER_W)], vals_v)

    zero16 = jnp.zeros((16,), jnp.float32)

    def _zero_body(i, c):
        ring[pl.ds(i * 16, 16)] = zero16
        return c

    lax.fori_loop(0, CHUNK_WORDS // 16, _zero_body, 0)

    pattern = lax.shift_left(
        lax.shift_right_logical(lax.iota(jnp.int32, 16), 1), 13)
    out_base = w * (TOK_PER_W * ROW_WORDS)

    def _body(j, c):
        idx = offs_v[pl.ds(j * 16, 16)] + pattern
        val = vals_v[pl.ds(j * 16, 16)]
        plsc.store_scatter(ring, [idx], val)
        pltpu.sync_copy(
            ring,
            out_hbm.at[pl.ds(out_base + j * CHUNK_WORDS, CHUNK_WORDS)])
        plsc.store_scatter(ring, [idx], zero16)
        return c

    lax.fori_loop(0, N_CHUNKS, _body, 0)


_NOISE = jax.random.gumbel(
    jax.random.key(42), (NUM_TOKENS, NUM_EXPERTS), dtype=jnp.float32)


def kernel(input, W):
    noise = _NOISE
    laux, le8, offs, vals = pl.pallas_call(
        _route_kernel,
        out_shape=[
            jax.ShapeDtypeStruct((1, 1), jnp.float32),
            jax.ShapeDtypeStruct((NUM_TOKENS, NUM_EXPERTS), jnp.int8),
            jax.ShapeDtypeStruct((NUM_TOKENS, 2), jnp.int32),
            jax.ShapeDtypeStruct((NUM_TOKENS, 2), jnp.float32),
        ],
    )(input, W, noise)
    disp = pl.pallas_call(
        _disp_kernel,
        grid=(NB,),
        in_specs=[
            pl.BlockSpec((TBLK, NUM_EXPERTS), lambda i: (i, 0)),
        ],
        out_specs=[
            pl.BlockSpec((TBLK, NUM_EXPERTS, CAPACITY), lambda i: (i, 0, 0)),
        ],
        out_shape=[
            jax.ShapeDtypeStruct((NUM_TOKENS, NUM_EXPERTS, CAPACITY),
                                 jnp.bool_),
        ],
    )(le8)[0]
    cw_flat = _sc_combine_fn()(offs.reshape(-1), vals.reshape(-1))
    cw = cw_flat.reshape(NUM_TOKENS, NUM_EXPERTS, CAPACITY)
    return laux.reshape(()), cw, disp


# speedup vs baseline: 1.5927x; 1.1310x over previous
import jax
import jax.numpy as jnp
from jax import lax
from jax.experimental import pallas as pl
from jax.experimental.pallas import tpu as pltpu

NUM_TOKENS = 4096
MODEL_DIM = 1024
NUM_EXPERTS = 64
CAPACITY = 2 * NUM_TOKENS // NUM_EXPERTS
TBLK = 128
NB = NUM_TOKENS // TBLK
CHUNK = 128

_NOISE = jax.random.gumbel(
    jax.random.key(42), (NUM_TOKENS, NUM_EXPERTS), dtype=jnp.float32)


def _cumsum_tokens(m):
    nb = NUM_TOKENS // CHUNK
    mb = m.reshape(nb, CHUNK, NUM_EXPERTS)
    ii = lax.broadcasted_iota(jnp.int32, (CHUNK, CHUNK), 0)
    jj = lax.broadcasted_iota(jnp.int32, (CHUNK, CHUNK), 1)
    tri = (jj <= ii).astype(jnp.float32)
    trib = jnp.broadcast_to(tri, (nb, CHUNK, CHUNK))
    within = lax.dot_general(
        trib, mb, (((2,), (1,)), ((0,), (0,))),
        preferred_element_type=jnp.float32)
    sums = within[:, CHUNK - 1, :]
    bi = lax.broadcasted_iota(jnp.int32, (nb, nb), 0)
    bj = lax.broadcasted_iota(jnp.int32, (nb, nb), 1)
    tri_x = (bj < bi).astype(jnp.float32)
    carry = lax.dot_general(
        tri_x, sums, (((1,), (0,)), ((), ())),
        preferred_element_type=jnp.float32)
    return (within + carry[:, None, :]).reshape(NUM_TOKENS, NUM_EXPERTS)


def _gate_kernel(x_ref, w_ref, noise_ref, laux_ref, cw_ref, disp_ref,
                 cwe_scr, loce_scr):
    i = pl.program_id(0)

    @pl.when(i == 0)
    def _():
        x = x_ref[...]
        w = w_ref[...]
        logits = lax.dot_general(
            x, w, (((1,), (1,)), ((), ())),
            preferred_element_type=jnp.float32)
        m = jnp.max(logits, axis=1, keepdims=True)
        ex = jnp.exp(logits - m)
        gates = ex / jnp.sum(ex, axis=1, keepdims=True)
        iota_e = lax.broadcasted_iota(
            jnp.int32, (NUM_TOKENS, NUM_EXPERTS), 1)
        e1 = jnp.argmax(logits, axis=1)
        oh1 = iota_e == e1[:, None]
        g1 = jnp.max(gates, axis=1)
        lx = jnp.where(oh1, -jnp.inf, logits + noise_ref[...])
        e2 = jnp.argmax(lx, axis=1)
        oh2 = iota_e == e2[:, None]
        g2 = jnp.sum(jnp.where(oh2, gates, 0.0), axis=1)

        cs1 = _cumsum_tokens(oh1.astype(jnp.float32))
        cs2 = _cumsum_tokens(oh2.astype(jnp.float32))
        counts1 = cs1[NUM_TOKENS - 1:NUM_TOKENS, :]
        loc1 = jnp.sum(jnp.where(oh1, cs1 - 1.0, 0.0), axis=1)
        loc2 = jnp.sum(jnp.where(oh2, cs2 - 1.0 + counts1, 0.0), axis=1)

        g1s = jnp.where(loc1 < CAPACITY, g1, 0.0)
        g2s = jnp.where(loc2 < CAPACITY, g2, 0.0)
        denom = jnp.maximum(g1s + g2s, jnp.finfo(jnp.float32).eps)
        g1n = g1s / denom
        g2n = g2s / denom

        cwe_scr[...] = (jnp.where(oh1, g1n[:, None], 0.0)
                        + jnp.where(oh2, g2n[:, None], 0.0))
        loce_scr[...] = (jnp.where(oh1, loc1[:, None], 0.0)
                         + jnp.where(oh2, loc2[:, None], 0.0)
                         ).astype(jnp.int32)

        me_sum = jnp.sum(gates, axis=0, keepdims=True)
        laux_ref[...] = jnp.sum(me_sum * counts1, axis=1, keepdims=True) / (
            float(NUM_EXPERTS) * NUM_TOKENS * NUM_TOKENS)

    cw = cwe_scr[pl.ds(i * TBLK, TBLK), :]
    loce = loce_scr[pl.ds(i * TBLK, TBLK), :]
    iota_c = lax.broadcasted_iota(
        jnp.int32, (TBLK, NUM_EXPERTS, CAPACITY), 2)
    out = jnp.where(iota_c == loce[:, :, None], cw[:, :, None], 0.0)
    cw_ref[...] = out
    disp_ref[...] = out != 0.0


def kernel(input, W):
    laux, cw, disp = pl.pallas_call(
        _gate_kernel,
        grid=(NB,),
        in_specs=[
            pl.BlockSpec((NUM_TOKENS, MODEL_DIM), lambda i: (0, 0)),
            pl.BlockSpec((NUM_EXPERTS, MODEL_DIM), lambda i: (0, 0)),
            pl.BlockSpec((NUM_TOKENS, NUM_EXPERTS), lambda i: (0, 0)),
        ],
        out_specs=[
            pl.BlockSpec((1, 1), lambda i: (0, 0)),
            pl.BlockSpec((TBLK, NUM_EXPERTS, CAPACITY), lambda i: (i, 0, 0)),
            pl.BlockSpec((TBLK, NUM_EXPERTS, CAPACITY), lambda i: (i, 0, 0)),
        ],
        out_shape=[
            jax.ShapeDtypeStruct((1, 1), jnp.float32),
            jax.ShapeDtypeStruct((NUM_TOKENS, NUM_EXPERTS, CAPACITY),
                                 jnp.float32),
            jax.ShapeDtypeStruct((NUM_TOKENS, NUM_EXPERTS, CAPACITY),
                                 jnp.bool_),
        ],
        scratch_shapes=[
            pltpu.VMEM((NUM_TOKENS, NUM_EXPERTS), jnp.float32),
            pltpu.VMEM((NUM_TOKENS, NUM_EXPERTS), jnp.int32),
        ],
    )(input, W, _NOISE)
    return laux.reshape(()), cw, disp
